# baseline (speedup 1.0000x reference)
import functools

import jax
import jax.numpy as jnp
from jax import lax
from jax.experimental import pallas as pl
from jax.experimental.pallas import tpu as pltpu
from jax.experimental.pallas import tpu_sc as plsc

NC = 2
NS = 16
NW = NC * NS
CHUNK = 80

BN = 2000
BE = 4000


def _seg_tc_body(v_ref, ef_ref, ones_ref, msg_ref, deg_ref):
    eb = pl.program_id(1)
    nb = pl.program_id(0)
    v_blk = v_ref[0, 0]
    ids = nb * BN + lax.broadcasted_iota(jnp.int32, (1, BN), 1)
    oh = (v_blk[:, None] == ids).astype(jnp.bfloat16)
    acc = jnp.dot(oh.T, ef_ref[...].astype(jnp.bfloat16),
                  preferred_element_type=jnp.float32)
    dacc = jnp.dot(oh.T, ones_ref[...],
                   preferred_element_type=jnp.float32)

    @pl.when(eb == 0)
    def _():
        msg_ref[...] = acc
        deg_ref[...] = dacc

    @pl.when(eb != 0)
    def _():
        msg_ref[...] += acc
        deg_ref[...] += dacc


def _seg_sum_tc(ef2, v, N, E):
    v3 = v.reshape(E // BE, 1, BE)
    ones8 = jnp.ones((BE, 8), jnp.bfloat16)
    return pl.pallas_call(
        _seg_tc_body,
        grid=(N // BN, E // BE),
        in_specs=[
            pl.BlockSpec((1, 1, BE), lambda nb, eb: (eb, 0, 0)),
            pl.BlockSpec((BE, 128), lambda nb, eb: (eb, 0)),
            pl.BlockSpec((BE, 8), lambda nb, eb: (0, 0)),
        ],
        out_specs=[
            pl.BlockSpec((BN, 128), lambda nb, eb: (nb, 0)),
            pl.BlockSpec((BN, 8), lambda nb, eb: (nb, 0)),
        ],
        out_shape=[
            jax.ShapeDtypeStruct((N, 128), jnp.float32),
            jax.ShapeDtypeStruct((N, 8), jnp.float32),
        ],
    )(v3, ef2, ones8)


def _gather_sc(h, u, v, N, E):
    D = h.shape[1]
    per_w = E // NW
    n_chunks = per_w // CHUNK
    mesh = plsc.VectorSubcoreMesh(core_axis_name="c", subcore_axis_name="s")

    @functools.partial(
        pl.kernel,
        out_type=(
            jax.ShapeDtypeStruct((E, D), jnp.float32),
            jax.ShapeDtypeStruct((E, D), jnp.float32),
        ),
        mesh=mesh,
        scratch_types=[
            pltpu.VMEM((CHUNK,), jnp.int32),
            pltpu.VMEM((CHUNK,), jnp.int32),
            pltpu.VMEM((CHUNK, D), jnp.float32),
            pltpu.VMEM((CHUNK, D), jnp.float32),
            pltpu.SemaphoreType.DMA,
            pltpu.SemaphoreType.DMA,
        ],
    )
    def k(h_hbm, u_hbm, v_hbm, gu_hbm, gv_hbm,
          idx_u, idx_w, rows_a, rows_b, sem1, sem2):
        cid = lax.axis_index("c")
        sid = lax.axis_index("s")
        wid = sid * NC + cid
        base = wid * per_w

        @pl.loop(0, n_chunks)
        def _(kk):
            off = base + kk * CHUNK
            pltpu.sync_copy(u_hbm.at[pl.ds(off, CHUNK)], idx_u)
            pltpu.sync_copy(v_hbm.at[pl.ds(off, CHUNK)], idx_w)
            cu = pltpu.async_copy(h_hbm.at[idx_u], rows_a, sem1)
            cv = pltpu.async_copy(h_hbm.at[idx_w], rows_b, sem2)
            cu.wait()
            cv.wait()
            pltpu.sync_copy(rows_a, gu_hbm.at[pl.ds(off, CHUNK)])
            pltpu.sync_copy(rows_b, gv_hbm.at[pl.ds(off, CHUNK)])

    return k(h, u, v)


def _node_mlp_body(nf_ref, msg_ref, deg_ref, wt_ref, b_ref, h_ref):
    deg = deg_ref[:, 0:1]
    hn = msg_ref[...] / jnp.maximum(deg, 1.0)
    wt = wt_ref[...]
    acc = jnp.dot(nf_ref[...], wt[:128], precision=lax.Precision.HIGHEST,
                  preferred_element_type=jnp.float32)
    acc += jnp.dot(hn, wt[128:], precision=lax.Precision.HIGHEST,
                   preferred_element_type=jnp.float32)
    h_ref[...] = jnp.maximum(acc + b_ref[...], 0.0)


def _node_mlp(nf2, msg, degw, W_apply_w, W_apply_b, N):
    R = 1000
    grid = (N // R,)
    wt = W_apply_w.T
    b = W_apply_b.reshape(1, -1)
    return pl.pallas_call(
        _node_mlp_body,
        grid=grid,
        in_specs=[
            pl.BlockSpec((R, 128), lambda i: (i, 0)),
            pl.BlockSpec((R, 128), lambda i: (i, 0)),
            pl.BlockSpec((R, 8), lambda i: (i, 0)),
            pl.BlockSpec((256, 128), lambda i: (0, 0)),
            pl.BlockSpec((1, 128), lambda i: (0, 0)),
        ],
        out_specs=pl.BlockSpec((R, 128), lambda i: (i, 0)),
        out_shape=jax.ShapeDtypeStruct((N, 128), jnp.float32),
    )(nf2, msg, degw, wt, b)


def _edge_mlp_body(gu_ref, gv_ref, wu_ref, wv_ref, b_ref, out_ref):
    acc = jnp.dot(gu_ref[...].astype(jnp.bfloat16), wu_ref[...],
                  preferred_element_type=jnp.float32)
    acc += jnp.dot(gv_ref[...].astype(jnp.bfloat16), wv_ref[...],
                   preferred_element_type=jnp.float32)
    out_ref[...] = acc + b_ref[...]


def _edge_mlp(gu, gv, W_edge_w, W_edge_b, E):
    B = 3200
    grid = (E // B,)
    wu = W_edge_w[:, :128].T.astype(jnp.bfloat16)
    wv = W_edge_w[:, 128:].T.astype(jnp.bfloat16)
    b = W_edge_b.reshape(1, -1)
    return pl.pallas_call(
        _edge_mlp_body,
        grid=grid,
        in_specs=[
            pl.BlockSpec((B, 128), lambda i: (i, 0)),
            pl.BlockSpec((B, 128), lambda i: (i, 0)),
            pl.BlockSpec((128, 256), lambda i: (0, 0)),
            pl.BlockSpec((128, 256), lambda i: (0, 0)),
            pl.BlockSpec((1, 256), lambda i: (0, 0)),
        ],
        out_specs=pl.BlockSpec((B, 256), lambda i: (i, 0)),
        out_shape=jax.ShapeDtypeStruct((E, 256), jnp.float32),
    )(gu, gv, wu, wv, b)


def kernel(nfeats, efeats, edge_index, W_apply_w, W_apply_b, W_edge_w, W_edge_b):
    N = nfeats.shape[0]
    E = efeats.shape[0]
    nf2 = nfeats.reshape(N, -1)
    ef2 = efeats.reshape(E, -1)
    u = edge_index[0]
    v = edge_index[1]

    msg, degw = _seg_sum_tc(ef2, v, N, E)
    h = _node_mlp(nf2, msg, degw, W_apply_w, W_apply_b, N)
    gu, gv = _gather_sc(h, u, v, N, E)
    edge = _edge_mlp(gu, gv, W_edge_w, W_edge_b, E)

    return h.reshape(N, 1, 128), edge.reshape(E, 1, 256)


# speedup vs baseline: 1.0342x; 1.0342x over previous
import functools

import jax
import jax.numpy as jnp
from jax import lax
from jax.experimental import pallas as pl
from jax.experimental.pallas import tpu as pltpu
from jax.experimental.pallas import tpu_sc as plsc

NC = 2
NS = 16
NW = NC * NS
CHUNK = 80

BN = 2000
BE = 4000


def _seg_tc_body(v_ref, ef_ref, ones_ref, msg_ref, deg_ref):
    eb = pl.program_id(1)
    nb = pl.program_id(0)
    v_blk = v_ref[0, 0]
    ids = nb * BN + lax.broadcasted_iota(jnp.int32, (1, BN), 1)
    oh = (v_blk[:, None] == ids).astype(jnp.bfloat16)
    acc = jnp.dot(oh.T, ef_ref[...].astype(jnp.bfloat16),
                  preferred_element_type=jnp.float32)
    dacc = jnp.dot(oh.T, ones_ref[...],
                   preferred_element_type=jnp.float32)

    @pl.when(eb == 0)
    def _():
        msg_ref[...] = acc
        deg_ref[...] = dacc

    @pl.when(eb != 0)
    def _():
        msg_ref[...] += acc
        deg_ref[...] += dacc


def _seg_sum_tc(ef2, v, N, E):
    v3 = v.reshape(E // BE, 1, BE)
    ones8 = jnp.ones((BE, 8), jnp.bfloat16)
    return pl.pallas_call(
        _seg_tc_body,
        grid=(N // BN, E // BE),
        in_specs=[
            pl.BlockSpec((1, 1, BE), lambda nb, eb: (eb, 0, 0)),
            pl.BlockSpec((BE, 128), lambda nb, eb: (eb, 0)),
            pl.BlockSpec((BE, 8), lambda nb, eb: (0, 0)),
        ],
        out_specs=[
            pl.BlockSpec((BN, 128), lambda nb, eb: (nb, 0)),
            pl.BlockSpec((BN, 8), lambda nb, eb: (nb, 0)),
        ],
        out_shape=[
            jax.ShapeDtypeStruct((N, 128), jnp.float32),
            jax.ShapeDtypeStruct((N, 8), jnp.float32),
        ],
    )(v3, ef2, ones8)


def _gather_sc(h, u, v, N, E):
    D = h.shape[1]
    per_w = E // NW
    n_chunks = per_w // CHUNK
    mesh = plsc.VectorSubcoreMesh(core_axis_name="c", subcore_axis_name="s")

    K = 5
    BATCH = CHUNK * K
    n_iters = per_w // BATCH

    @functools.partial(
        pl.kernel,
        out_type=(
            jax.ShapeDtypeStruct((E, D), jnp.float32),
            jax.ShapeDtypeStruct((E, D), jnp.float32),
        ),
        mesh=mesh,
        scratch_types=[
            pltpu.VMEM((BATCH,), jnp.int32),
            pltpu.VMEM((BATCH,), jnp.int32),
            pltpu.VMEM((BATCH, D), jnp.float32),
            pltpu.VMEM((BATCH, D), jnp.float32),
            pltpu.SemaphoreType.DMA,
            pltpu.SemaphoreType.DMA,
        ],
    )
    def k(h_hbm, u_hbm, v_hbm, gu_hbm, gv_hbm,
          idx_u, idx_w, rows_a, rows_b, sem1, sem2):
        cid = lax.axis_index("c")
        sid = lax.axis_index("s")
        wid = sid * NC + cid
        base = wid * per_w

        @pl.loop(0, n_iters)
        def _(kk):
            off = base + kk * BATCH
            pltpu.sync_copy(u_hbm.at[pl.ds(off, BATCH)], idx_u)
            pltpu.sync_copy(v_hbm.at[pl.ds(off, BATCH)], idx_w)
            copies = []
            for j in range(K):
                sl = pl.ds(j * CHUNK, CHUNK)
                copies.append(pltpu.async_copy(
                    h_hbm.at[idx_u.at[sl]], rows_a.at[sl], sem1))
                copies.append(pltpu.async_copy(
                    h_hbm.at[idx_w.at[sl]], rows_b.at[sl], sem2))
            for c in copies:
                c.wait()
            pltpu.sync_copy(rows_a, gu_hbm.at[pl.ds(off, BATCH)])
            pltpu.sync_copy(rows_b, gv_hbm.at[pl.ds(off, BATCH)])

    return k(h, u, v)


def _node_mlp_body(nf_ref, msg_ref, deg_ref, wt_ref, b_ref, h_ref):
    deg = deg_ref[:, 0:1]
    hn = msg_ref[...] / jnp.maximum(deg, 1.0)
    wt = wt_ref[...]
    acc = jnp.dot(nf_ref[...], wt[:128], precision=lax.Precision.HIGHEST,
                  preferred_element_type=jnp.float32)
    acc += jnp.dot(hn, wt[128:], precision=lax.Precision.HIGHEST,
                   preferred_element_type=jnp.float32)
    h_ref[...] = jnp.maximum(acc + b_ref[...], 0.0)


def _node_mlp(nf2, msg, degw, W_apply_w, W_apply_b, N):
    R = 1000
    grid = (N // R,)
    wt = W_apply_w.T
    b = W_apply_b.reshape(1, -1)
    return pl.pallas_call(
        _node_mlp_body,
        grid=grid,
        in_specs=[
            pl.BlockSpec((R, 128), lambda i: (i, 0)),
            pl.BlockSpec((R, 128), lambda i: (i, 0)),
            pl.BlockSpec((R, 8), lambda i: (i, 0)),
            pl.BlockSpec((256, 128), lambda i: (0, 0)),
            pl.BlockSpec((1, 128), lambda i: (0, 0)),
        ],
        out_specs=pl.BlockSpec((R, 128), lambda i: (i, 0)),
        out_shape=jax.ShapeDtypeStruct((N, 128), jnp.float32),
    )(nf2, msg, degw, wt, b)


def _edge_mlp_body(gu_ref, gv_ref, wu_ref, wv_ref, b_ref, out_ref):
    acc = jnp.dot(gu_ref[...].astype(jnp.bfloat16), wu_ref[...],
                  preferred_element_type=jnp.float32)
    acc += jnp.dot(gv_ref[...].astype(jnp.bfloat16), wv_ref[...],
                   preferred_element_type=jnp.float32)
    out_ref[...] = acc + b_ref[...]


def _edge_mlp(gu, gv, W_edge_w, W_edge_b, E):
    B = 3200
    grid = (E // B,)
    wu = W_edge_w[:, :128].T.astype(jnp.bfloat16)
    wv = W_edge_w[:, 128:].T.astype(jnp.bfloat16)
    b = W_edge_b.reshape(1, -1)
    return pl.pallas_call(
        _edge_mlp_body,
        grid=grid,
        in_specs=[
            pl.BlockSpec((B, 128), lambda i: (i, 0)),
            pl.BlockSpec((B, 128), lambda i: (i, 0)),
            pl.BlockSpec((128, 256), lambda i: (0, 0)),
            pl.BlockSpec((128, 256), lambda i: (0, 0)),
            pl.BlockSpec((1, 256), lambda i: (0, 0)),
        ],
        out_specs=pl.BlockSpec((B, 256), lambda i: (i, 0)),
        out_shape=jax.ShapeDtypeStruct((E, 256), jnp.float32),
    )(gu, gv, wu, wv, b)


def kernel(nfeats, efeats, edge_index, W_apply_w, W_apply_b, W_edge_w, W_edge_b):
    N = nfeats.shape[0]
    E = efeats.shape[0]
    nf2 = nfeats.reshape(N, -1)
    ef2 = efeats.reshape(E, -1)
    u = edge_index[0]
    v = edge_index[1]

    msg, degw = _seg_sum_tc(ef2, v, N, E)
    h = _node_mlp(nf2, msg, degw, W_apply_w, W_apply_b, N)
    gu, gv = _gather_sc(h, u, v, N, E)
    edge = _edge_mlp(gu, gv, W_edge_w, W_edge_b, E)

    return h.reshape(N, 1, 128), edge.reshape(E, 1, 256)
